# baseline (speedup 1.0000x reference)
import jax
import jax.numpy as jnp
from jax import lax
from jax.experimental import pallas as pl
from jax.experimental.pallas import tpu as pltpu
from jax.experimental.pallas import tpu_sc as plsc

_C = 1024
_NBUF = 4


def _sc_lookup_body(gid_hbm, table_hbm, vec_hbm, gid_smem):
    c = lax.axis_index("c")

    @pl.when(c == 0)
    def _():
        pltpu.sync_copy(gid_hbm, gid_smem)
        g = gid_smem[0]
        pltpu.sync_copy(table_hbm.at[pl.ds(g, 1)], vec_hbm)


def _tc_add_body(vec_ref, tok_hbm, out_hbm, *scratch):
    in_bufs = scratch[:_NBUF]
    out_bufs = scratch[_NBUF:2 * _NBUF]
    in_sems = scratch[2 * _NBUF:3 * _NBUF]
    out_sems = scratch[3 * _NBUF:4 * _NBUF]
    rows = tok_hbm.shape[0]
    nchunk = rows // _C
    vec = vec_ref[0, :]

    for b in range(_NBUF):
        pltpu.make_async_copy(
            tok_hbm.at[pl.ds(b * _C, _C)], in_bufs[b], in_sems[b]).start()

    def _step(g, b):
        pltpu.make_async_copy(
            tok_hbm.at[pl.ds(0, _C)], in_bufs[b], in_sems[b]).wait()

        @pl.when(g >= _NBUF)
        def _():
            pltpu.make_async_copy(
                out_bufs[b], out_hbm.at[pl.ds(0, _C)], out_sems[b]).wait()

        out_bufs[b][...] = in_bufs[b][...] + vec[None, :]

        @pl.when(g + _NBUF < nchunk)
        def _():
            pltpu.make_async_copy(
                tok_hbm.at[pl.ds((g + _NBUF) * _C, _C)],
                in_bufs[b], in_sems[b]).start()

        pltpu.make_async_copy(
            out_bufs[b], out_hbm.at[pl.ds(g * _C, _C)], out_sems[b]).start()

    def _outer(i, carry):
        for b in range(_NBUF):
            _step(i * _NBUF + b, b)
        return carry

    lax.fori_loop(0, nchunk // _NBUF, _outer, 0)

    for b in range(_NBUF):
        pltpu.make_async_copy(
            out_bufs[b], out_hbm.at[pl.ds(0, _C)], out_sems[b]).wait()


def kernel(tokens, group_id, group_id_vecs):
    b, s, d = tokens.shape
    rows = b * s
    tok2d = tokens.reshape(rows, d)
    gid = jnp.asarray(group_id, jnp.int32).reshape((1,))

    sc_lookup = pl.kernel(
        _sc_lookup_body,
        out_type=jax.ShapeDtypeStruct((1, d), jnp.float32),
        mesh=plsc.ScalarSubcoreMesh(axis_name="c", num_cores=2),
        scratch_types=[
            pltpu.SMEM((1,), jnp.int32),
        ],
    )
    vec = sc_lookup(gid, group_id_vecs)

    out = pl.pallas_call(
        _tc_add_body,
        grid=(1,),
        in_specs=[
            pl.BlockSpec(memory_space=pltpu.VMEM),
            pl.BlockSpec(memory_space=pltpu.HBM),
        ],
        out_specs=pl.BlockSpec(memory_space=pltpu.HBM),
        scratch_shapes=(
            [pltpu.VMEM((_C, d), jnp.float32)] * (2 * _NBUF)
            + [pltpu.SemaphoreType.DMA] * (2 * _NBUF)
        ),
        out_shape=jax.ShapeDtypeStruct((rows, d), tokens.dtype),
    )(vec, tok2d)
    return out.reshape(b, s, d)


# speedup vs baseline: 1.0146x; 1.0146x over previous
import jax
import jax.numpy as jnp
from jax import lax
from jax.experimental import pallas as pl
from jax.experimental.pallas import tpu as pltpu
from jax.experimental.pallas import tpu_sc as plsc

_BM = 2048


def _sc_lookup_body(gid_hbm, table_hbm, vec_hbm, gid_smem):
    c = lax.axis_index("c")

    @pl.when(c == 0)
    def _():
        pltpu.sync_copy(gid_hbm, gid_smem)
        g = gid_smem[0]
        pltpu.sync_copy(table_hbm.at[pl.ds(g, 1)], vec_hbm)


def _add_kernel(vec_ref, tok_ref, out_ref):
    out_ref[...] = tok_ref[...] + vec_ref[...]


def kernel(tokens, group_id, group_id_vecs):
    b, s, d = tokens.shape
    rows = b * s
    tok2d = tokens.reshape(rows, d)
    gid = jnp.asarray(group_id, jnp.int32).reshape((1,))

    sc_lookup = pl.kernel(
        _sc_lookup_body,
        out_type=jax.ShapeDtypeStruct((1, d), jnp.float32),
        mesh=plsc.ScalarSubcoreMesh(axis_name="c", num_cores=2),
        scratch_types=[
            pltpu.SMEM((1,), jnp.int32),
        ],
    )
    vec = sc_lookup(gid, group_id_vecs)

    out = pl.pallas_call(
        _add_kernel,
        grid=(rows // _BM,),
        in_specs=[
            pl.BlockSpec((1, d), lambda i: (0, 0)),
            pl.BlockSpec((_BM, d), lambda i: (i, 0)),
        ],
        out_specs=pl.BlockSpec((_BM, d), lambda i: (i, 0)),
        out_shape=jax.ShapeDtypeStruct((rows, d), tokens.dtype),
        compiler_params=pltpu.CompilerParams(
            dimension_semantics=("parallel",),
        ),
    )(vec, tok2d)
    return out.reshape(b, s, d)
